# baseline (speedup 1.0000x reference)
import functools
import math

import jax
import jax.numpy as jnp
from jax import lax
from jax.experimental import pallas as pl
from jax.experimental.pallas import tpu as pltpu

_LARGE_NUMBER = 1e10
_NEG = -1e30
_K = 32
_S = 2
_ROWS_PER_BLOCK = 1024


def _laep(x1, x2):
    amax = lax.max(x1, x2)
    delta = lax.sub(x1, x2)
    return lax.add(amax, lax.log1p(lax.exp(lax.neg(lax.abs(delta)))))


def _simple_body(th_ref, u_ref, marg_ref, masks_ref, qscr, *, n, kp1):
    f32 = jnp.float32
    neg_row = jnp.full((1, 8, 128), _NEG, f32)

    b = jnp.zeros((1, 8, 128), f32)
    for i in range(n - 1, _K - 1, -1):
        th_i = th_ref[pl.ds(i, 1), 0]
        shifted = jnp.concatenate([neg_row, b], axis=0)
        bnext_ext = jnp.concatenate([b, neg_row], axis=0)
        lognum = th_i + shifted
        b = _laep(bnext_ext, lognum)
        qscr[i, : b.shape[0]] = jnp.exp(lognum - b)

    for i in range(_K - 1, -1, -1):
        th_i = th_ref[pl.ds(i, 1), 0]
        lognum = th_i + b[:-1]
        bnew = _laep(b[1:], lognum)
        qscr[i, _K - i : _K + 1] = jnp.exp(lognum - bnew)
        b = bnew

    zero_row = jnp.zeros((1, 8, 128), f32)
    pi_v = jnp.ones((1, 8, 128), f32)
    rs = [jnp.full((8, 128), _K, jnp.int32) for _ in range(_S)]
    for i in range(n):
        lo = max(0, _K - i)
        hi = min(_K, n - i)
        w = hi - lo + 1
        qi = qscr[i, lo : hi + 1]
        t = pi_v * qi
        marg_i = jnp.sum(t, axis=0)
        marg_ref[pl.ds(i, 1)] = marg_i[None, None]
        base = pi_v - t
        tup = zero_row if w == 1 else jnp.concatenate([t[1:], zero_row], axis=0)
        lo2 = max(0, _K - i - 1)
        hi2 = min(_K, n - i - 1)
        if lo2 < lo:
            pi_v = jnp.concatenate([t[:1], base + tup], axis=0)
        else:
            pi_v = (base + tup)[: hi2 - lo2 + 1]
        for s in range(_S):
            r = rs[s]
            x = r - lo
            vals = [qi[j] for j in range(w)]
            level = 0
            while len(vals) > 1:
                bit = (x & (1 << level)) != 0
                vals = [
                    jnp.where(bit, vals[2 * m + 1], vals[2 * m])
                    if 2 * m + 1 < len(vals) else vals[2 * m]
                    for m in range((len(vals) + 1) // 2)
                ]
                level += 1
            p = vals[0]
            u = u_ref[pl.ds(_S * i + s, 1), 0][0]
            inc = u < p
            st = (inc.astype(f32) - marg_i) + marg_i
            masks_ref[pl.ds(s * n + i, 1)] = st[None, None]
            rs[s] = r - inc.astype(jnp.int32)


def kernel(scores):
    nnodes = scores.shape[0]
    if nnodes % 2 == 0 and nnodes >= 2048:
        h = nnodes // 2
        u_all = jax.random.uniform(
            jax.random.key(1),
            (2 ** int(math.ceil(math.log2(scores.shape[1]))), _S,
             nnodes * scores.shape[2]),
            dtype=scores.dtype)
        hr = h * scores.shape[2]
        s0, m0 = _half(scores[:h], u_all[:, :, :hr])
        s1, m1 = _half(scores[h:], u_all[:, :, hr:])
        return (jnp.concatenate([s0, s1], axis=1),
                jnp.concatenate([m0, m1], axis=0))
    n0 = 2 ** int(math.ceil(math.log2(scores.shape[1])))
    u_all = jax.random.uniform(
        jax.random.key(1), (n0, _S, nnodes * scores.shape[2]),
        dtype=scores.dtype)
    return _half(scores, u_all)


def _half(scores, u):
    nnodes, choices, ensemble = scores.shape
    local_k = min(_K, choices)
    kp1 = local_k + 1
    n = 2 ** int(math.ceil(math.log2(choices)))
    rows = nnodes * ensemble
    rpb = _ROWS_PER_BLOCK
    nblocks = (rows + rpb - 1) // rpb
    rows_pad = nblocks * rpb

    th = jnp.transpose(scores, (1, 0, 2)).reshape(choices, rows)
    if n > choices:
        th = jnp.concatenate(
            [th, jnp.full((n - choices, rows), -_LARGE_NUMBER, th.dtype)], axis=0)
    th4 = jnp.pad(th, ((0, 0), (0, rows_pad - rows))).reshape(n, nblocks, 8, 128)

    u4 = jnp.pad(u.reshape(n * _S, rows), ((0, 0), (0, rows_pad - rows)))
    u4 = u4.reshape(n * _S, nblocks, 8, 128)

    body = functools.partial(_simple_body, n=n, kp1=kp1)
    marg4, masks4 = pl.pallas_call(
        body,
        grid=(nblocks,),
        in_specs=[
            pl.BlockSpec((n, 1, 8, 128), lambda g: (0, g, 0, 0)),
            pl.BlockSpec((n * _S, 1, 8, 128), lambda g: (0, g, 0, 0)),
        ],
        out_specs=[
            pl.BlockSpec((n, 1, 8, 128), lambda g: (0, g, 0, 0)),
            pl.BlockSpec((_S * n, 1, 8, 128), lambda g: (0, g, 0, 0)),
        ],
        out_shape=[
            jax.ShapeDtypeStruct((n, nblocks, 8, 128), jnp.float32),
            jax.ShapeDtypeStruct((_S * n, nblocks, 8, 128), jnp.float32),
        ],
        scratch_shapes=[
            pltpu.VMEM((n, kp1, 8, 128), jnp.float32),
        ],
    )(th4, u4)

    marg = marg4.reshape(n, rows_pad)[:choices, :rows]
    marginals = marg.reshape(choices, nnodes, ensemble).transpose(1, 0, 2)

    masks = masks4.reshape(_S, n, rows_pad)[:, :choices, :rows]
    samples = masks.reshape(_S, choices, nnodes, ensemble).transpose(0, 2, 1, 3)
    return samples, marginals


# speedup vs baseline: 1.3706x; 1.3706x over previous
import functools
import math

import jax
import jax.numpy as jnp
from jax import lax
from jax.experimental import pallas as pl
from jax.experimental.pallas import tpu as pltpu

_LARGE_NUMBER = 1e10
_NEG = -1e30
_K = 32
_S = 2
_ROWS_PER_BLOCK = 2048
_SUB = _ROWS_PER_BLOCK // 128


def _laep(x1, x2):
    amax = lax.max(x1, x2)
    delta = lax.sub(x1, x2)
    return lax.add(amax, lax.log1p(lax.exp(lax.neg(lax.abs(delta)))))


def _simple_body(th_ref, u_ref, marg_ref, masks_ref, qscr, *, n, kp1):
    f32 = jnp.float32
    neg_row = jnp.full((1, _SUB, 128), _NEG, f32)

    b = jnp.zeros((1, _SUB, 128), f32)
    for i in range(n - 1, _K - 1, -1):
        th_i = th_ref[pl.ds(i, 1), 0]
        shifted = jnp.concatenate([neg_row, b], axis=0)
        bnext_ext = jnp.concatenate([b, neg_row], axis=0)
        lognum = th_i + shifted
        b = _laep(bnext_ext, lognum)
        qscr[i, : b.shape[0]] = jnp.exp(lognum - b)

    for i in range(_K - 1, -1, -1):
        th_i = th_ref[pl.ds(i, 1), 0]
        lognum = th_i + b[:-1]
        bnew = _laep(b[1:], lognum)
        qscr[i, _K - i : _K + 1] = jnp.exp(lognum - bnew)
        b = bnew

    zero_row = jnp.zeros((1, _SUB, 128), f32)
    pi_v = jnp.ones((1, _SUB, 128), f32)
    rs = [jnp.full((_SUB, 128), _K, jnp.int32) for _ in range(_S)]
    for i in range(n):
        lo = max(0, _K - i)
        hi = min(_K, n - i)
        w = hi - lo + 1
        qi = qscr[i, lo : hi + 1]
        t = pi_v * qi
        marg_i = jnp.sum(t, axis=0)
        marg_ref[pl.ds(i, 1)] = marg_i[None, None]
        base = pi_v - t
        tup = zero_row if w == 1 else jnp.concatenate([t[1:], zero_row], axis=0)
        lo2 = max(0, _K - i - 1)
        hi2 = min(_K, n - i - 1)
        if lo2 < lo:
            pi_v = jnp.concatenate([t[:1], base + tup], axis=0)
        else:
            pi_v = (base + tup)[: hi2 - lo2 + 1]
        for s in range(_S):
            r = rs[s]
            x = r - lo
            vals = [qi[j] for j in range(w)]
            level = 0
            while len(vals) > 1:
                bit = (x & (1 << level)) != 0
                vals = [
                    jnp.where(bit, vals[2 * m + 1], vals[2 * m])
                    if 2 * m + 1 < len(vals) else vals[2 * m]
                    for m in range((len(vals) + 1) // 2)
                ]
                level += 1
            p = vals[0]
            u = u_ref[pl.ds(_S * i + s, 1), 0][0]
            inc = u < p
            st = (inc.astype(f32) - marg_i) + marg_i
            masks_ref[pl.ds(s * n + i, 1)] = st[None, None]
            rs[s] = r - inc.astype(jnp.int32)


def kernel(scores):
    nnodes, choices, ensemble = scores.shape
    local_k = min(_K, choices)
    kp1 = local_k + 1
    n = 2 ** int(math.ceil(math.log2(choices)))
    rows = nnodes * ensemble
    rpb = _ROWS_PER_BLOCK
    nblocks = (rows + rpb - 1) // rpb
    rows_pad = nblocks * rpb

    th = jnp.transpose(scores, (1, 0, 2)).reshape(choices, rows)
    if n > choices:
        th = jnp.concatenate(
            [th, jnp.full((n - choices, rows), -_LARGE_NUMBER, th.dtype)], axis=0)
    th4 = jnp.pad(th, ((0, 0), (0, rows_pad - rows))).reshape(n, nblocks, _SUB, 128)

    u = jax.random.uniform(jax.random.key(1), (n, _S, rows), dtype=scores.dtype)
    u4 = jnp.pad(u.reshape(n * _S, rows), ((0, 0), (0, rows_pad - rows)))
    u4 = u4.reshape(n * _S, nblocks, _SUB, 128)

    body = functools.partial(_simple_body, n=n, kp1=kp1)
    marg4, masks4 = pl.pallas_call(
        body,
        grid=(nblocks,),
        in_specs=[
            pl.BlockSpec((n, 1, _SUB, 128), lambda g: (0, g, 0, 0)),
            pl.BlockSpec((n * _S, 1, _SUB, 128), lambda g: (0, g, 0, 0)),
        ],
        out_specs=[
            pl.BlockSpec((n, 1, _SUB, 128), lambda g: (0, g, 0, 0)),
            pl.BlockSpec((_S * n, 1, _SUB, 128), lambda g: (0, g, 0, 0)),
        ],
        out_shape=[
            jax.ShapeDtypeStruct((n, nblocks, _SUB, 128), jnp.float32),
            jax.ShapeDtypeStruct((_S * n, nblocks, _SUB, 128), jnp.float32),
        ],
        scratch_shapes=[
            pltpu.VMEM((n, kp1, _SUB, 128), jnp.float32),
        ],
    )(th4, u4)

    marg = marg4.reshape(n, rows_pad)[:choices, :rows]
    marginals = marg.reshape(choices, nnodes, ensemble).transpose(1, 0, 2)

    masks = masks4.reshape(_S, n, rows_pad)[:, :choices, :rows]
    samples = masks.reshape(_S, choices, nnodes, ensemble).transpose(0, 2, 1, 3)
    return samples, marginals


# speedup vs baseline: 1.3778x; 1.0052x over previous
import functools
import math

import jax
import jax.numpy as jnp
from jax import lax
from jax.experimental import pallas as pl
from jax.experimental.pallas import tpu as pltpu

_LARGE_NUMBER = 1e10
_NEG = -1e30
_K = 32
_S = 2
_ROWS_PER_BLOCK = 1024


def _laep(x1, x2):
    amax = lax.max(x1, x2)
    delta = lax.sub(x1, x2)
    return lax.add(amax, lax.log1p(lax.exp(lax.neg(lax.abs(delta)))))


def _simple_body(th_ref, u_ref, marg_ref, masks_ref, qscr, *, n, kp1):
    f32 = jnp.float32
    neg_row = jnp.full((1, 8, 128), _NEG, f32)

    b = jnp.zeros((1, 8, 128), f32)
    for i in range(n - 1, _K - 1, -1):
        th_i = th_ref[pl.ds(i, 1), 0]
        shifted = jnp.concatenate([neg_row, b], axis=0)
        bnext_ext = jnp.concatenate([b, neg_row], axis=0)
        lognum = th_i + shifted
        b = _laep(bnext_ext, lognum)
        qscr[i, : b.shape[0]] = jnp.exp(lognum - b)

    for i in range(_K - 1, -1, -1):
        th_i = th_ref[pl.ds(i, 1), 0]
        lognum = th_i + b[:-1]
        bnew = _laep(b[1:], lognum)
        qscr[i, _K - i : _K + 1] = jnp.exp(lognum - bnew)
        b = bnew

    zero_row = jnp.zeros((1, 8, 128), f32)
    pi_v = jnp.ones((1, 8, 128), f32)
    rs = [jnp.full((8, 128), _K, jnp.int32) for _ in range(_S)]
    for i in range(n):
        lo = max(0, _K - i)
        hi = min(_K, n - i)
        w = hi - lo + 1
        qi = qscr[i, lo : hi + 1]
        t = pi_v * qi
        marg_i = jnp.sum(t, axis=0)
        marg_ref[pl.ds(i, 1)] = marg_i[None, None]
        base = pi_v - t
        tup = zero_row if w == 1 else jnp.concatenate([t[1:], zero_row], axis=0)
        lo2 = max(0, _K - i - 1)
        hi2 = min(_K, n - i - 1)
        if lo2 < lo:
            pi_v = jnp.concatenate([t[:1], base + tup], axis=0)
        else:
            pi_v = (base + tup)[: hi2 - lo2 + 1]
        for s in range(_S):
            r = rs[s]
            x = r - lo
            vals = [qi[j] for j in range(w)]
            level = 0
            while len(vals) > 1:
                bit = (x & (1 << level)) != 0
                vals = [
                    jnp.where(bit, vals[2 * m + 1], vals[2 * m])
                    if 2 * m + 1 < len(vals) else vals[2 * m]
                    for m in range((len(vals) + 1) // 2)
                ]
                level += 1
            p = vals[0]
            u = u_ref[pl.ds(_S * i + s, 1), 0][0]
            inc = u < p
            st = (inc.astype(f32) - marg_i) + marg_i
            masks_ref[pl.ds(s * n + i, 1)] = st[None, None]
            rs[s] = r - inc.astype(jnp.int32)


def kernel(scores):
    nnodes, choices, ensemble = scores.shape
    local_k = min(_K, choices)
    kp1 = local_k + 1
    n = 2 ** int(math.ceil(math.log2(choices)))
    rows = nnodes * ensemble
    rpb = _ROWS_PER_BLOCK
    nblocks = (rows + rpb - 1) // rpb
    rows_pad = nblocks * rpb

    th = jnp.transpose(scores, (1, 0, 2)).reshape(choices, rows)
    if n > choices:
        th = jnp.concatenate(
            [th, jnp.full((n - choices, rows), -_LARGE_NUMBER, th.dtype)], axis=0)
    th4 = jnp.pad(th, ((0, 0), (0, rows_pad - rows))).reshape(n, nblocks, 8, 128)

    u = jax.random.uniform(jax.random.key(1), (n, _S, rows), dtype=scores.dtype)
    u4 = jnp.pad(u.reshape(n * _S, rows), ((0, 0), (0, rows_pad - rows)))
    u4 = u4.reshape(n * _S, nblocks, 8, 128)

    body = functools.partial(_simple_body, n=n, kp1=kp1)
    marg4, masks4 = pl.pallas_call(
        body,
        grid=(nblocks,),
        in_specs=[
            pl.BlockSpec((n, 1, 8, 128), lambda g: (0, g, 0, 0)),
            pl.BlockSpec((n * _S, 1, 8, 128), lambda g: (0, g, 0, 0)),
        ],
        out_specs=[
            pl.BlockSpec((n, 1, 8, 128), lambda g: (0, g, 0, 0)),
            pl.BlockSpec((_S * n, 1, 8, 128), lambda g: (0, g, 0, 0)),
        ],
        out_shape=[
            jax.ShapeDtypeStruct((n, nblocks, 8, 128), jnp.float32),
            jax.ShapeDtypeStruct((_S * n, nblocks, 8, 128), jnp.float32),
        ],
        scratch_shapes=[
            pltpu.VMEM((n, kp1, 8, 128), jnp.float32),
        ],
    )(th4, u4)

    marg = marg4.reshape(n, rows_pad)[:choices, :rows]
    marginals = marg.reshape(choices, nnodes, ensemble).transpose(1, 0, 2)

    masks = masks4.reshape(_S, n, rows_pad)[:, :choices, :rows]
    samples = masks.reshape(_S, choices, nnodes, ensemble).transpose(0, 2, 1, 3)
    return samples, marginals
